# Initial kernel scaffold; baseline (speedup 1.0000x reference)
import jax
import jax.numpy as jnp
from jax.experimental import pallas as pl


def kernel(tokens, table):
    raise NotImplementedError("write your pallas kernel here")


# speedup vs baseline: 2.2926x; 2.2926x over previous
import math

import numpy as np
import jax
import jax.numpy as jnp
from jax import lax
from jax.experimental import pallas as pl
from jax.experimental.pallas import tpu as pltpu
from jax.experimental.pallas import tpu_sc as plsc

VOCAB = 100000
D_MODEL = 768
MAX_LEN = 8192
SEQ = 4096
BATCH = 4
SCALE = math.sqrt(D_MODEL)

NC, NS = 2, 16
NW = NC * NS
B_FLAT = BATCH * SEQ
B_PER_W = B_FLAT // NW
K = 64
N_CHUNKS = B_PER_W // K
LANES = 16
VECS = D_MODEL // LANES


def _make_pe(seq_len: int, d_model: int) -> np.ndarray:
    position = np.arange(0, seq_len, dtype=np.float32)[:, None]
    div_term = np.exp(
        np.arange(0, d_model, 2).astype(np.float32) * (-math.log(10000.0) / d_model)
    )
    pe = np.zeros((seq_len, d_model), dtype=np.float32)
    pe[:, 0::2] = np.sin(position * div_term)
    pe[:, 1::2] = np.cos(position * div_term)
    return pe


_PE = _make_pe(SEQ, D_MODEL)


def _body(idx_hbm, table_hbm, pe_hbm, out_hbm, idx_v, rows_v, pe_v, sem):
    c = lax.axis_index("c")
    s = lax.axis_index("s")
    wid = s * NC + c
    base = wid * B_PER_W
    pbase = lax.rem(base, SEQ)

    pltpu.sync_copy(idx_hbm.at[wid], idx_v)

    for i in range(N_CHUNKS):
        pltpu.sync_copy(pe_hbm.at[pl.ds(pbase + i * K, K)], pe_v)
        pltpu.async_copy(table_hbm.at[idx_v.at[i]], rows_v, sem).wait()

        def row_fma(r, carry):
            for j in range(VECS):
                sl = pl.ds(j * LANES, LANES)
                rows_v[r, sl] = rows_v[r, sl] * SCALE + pe_v[r, sl]
            return carry

        lax.fori_loop(0, K, row_fma, 0)

        pltpu.sync_copy(rows_v, out_hbm.at[pl.ds(base + i * K, K)])


def kernel(tokens, table):
    idx = tokens.reshape(NW, N_CHUNKS, K).astype(jnp.int32)
    pe = jnp.asarray(_PE)

    mesh = plsc.VectorSubcoreMesh(core_axis_name="c", subcore_axis_name="s")
    out = pl.kernel(
        _body,
        out_type=jax.ShapeDtypeStruct((B_FLAT, D_MODEL), jnp.float32),
        mesh=mesh,
        scratch_types=[
            pltpu.VMEM((N_CHUNKS, K), jnp.int32),
            pltpu.VMEM((K, D_MODEL), jnp.float32),
            pltpu.VMEM((K, D_MODEL), jnp.float32),
            pltpu.SemaphoreType.DMA,
        ],
    )(idx, table, pe)
    return out.reshape(BATCH, SEQ, D_MODEL)


# speedup vs baseline: 3.2615x; 1.4226x over previous
import math

import numpy as np
import jax
import jax.numpy as jnp
from jax import lax
from jax.experimental import pallas as pl
from jax.experimental.pallas import tpu as pltpu
from jax.experimental.pallas import tpu_sc as plsc

VOCAB = 100000
D_MODEL = 768
SEQ = 4096
BATCH = 4
SCALE = math.sqrt(D_MODEL)

NC, NS = 2, 16
NW = NC * NS
P_PER_W = SEQ // NW
KP = 32
NPC = P_PER_W // KP
NSTEP = NPC * BATCH
LANES = 16
VECS = D_MODEL // LANES


def _make_pe(seq_len: int, d_model: int) -> np.ndarray:
    position = np.arange(0, seq_len, dtype=np.float32)[:, None]
    div_term = np.exp(
        np.arange(0, d_model, 2).astype(np.float32) * (-math.log(10000.0) / d_model)
    )
    pe = np.zeros((seq_len, d_model), dtype=np.float32)
    pe[:, 0::2] = np.sin(position * div_term)
    pe[:, 1::2] = np.cos(position * div_term)
    return pe


_PE = _make_pe(SEQ, D_MODEL)


def _body(idx_hbm, table_hbm, pe_hbm, out_hbm,
          idx_v, r0, r1, p0, p1, gs0, gs1, os0, os1, ps0, ps1):
    rows = (r0, r1)
    pes = (p0, p1)
    gsem = (gs0, gs1)
    osem = (os0, os1)
    psem = (ps0, ps1)

    c = lax.axis_index("c")
    s = lax.axis_index("s")
    wid = s * NC + c
    posbase = wid * P_PER_W

    pltpu.sync_copy(idx_hbm.at[wid], idx_v)

    def start_pe(pc):
        return pltpu.async_copy(
            pe_hbm.at[pl.ds(posbase + pc * KP, KP)], pes[pc % 2], psem[pc % 2])

    def start_gather(k):
        pc, b = divmod(k, BATCH)
        return pltpu.async_copy(
            table_hbm.at[idx_v.at[pc, b]], rows[k % 2], gsem[k % 2])

    def start_out(k):
        pc, b = divmod(k, BATCH)
        dst = out_hbm.at[pl.ds(b * SEQ + posbase + pc * KP, KP)]
        return pltpu.async_copy(rows[k % 2], dst, osem[k % 2])

    pend_pe = [start_pe(0), start_pe(1)]
    pend_out = [None, None]
    pend_g = [None, None]
    pend_g[0] = start_gather(0)

    for k in range(NSTEP):
        buf = k % 2
        pc, b = divmod(k, BATCH)
        if k + 1 < NSTEP:
            nbuf = 1 - buf
            if pend_out[nbuf] is not None:
                pend_out[nbuf].wait()
                pend_out[nbuf] = None
            pend_g[nbuf] = start_gather(k + 1)

        pend_g[buf].wait()
        if b == 0 and pend_pe[pc % 2] is not None:
            pend_pe[pc % 2].wait()
            pend_pe[pc % 2] = None

        rv = rows[buf]
        pv = pes[pc % 2]

        @plsc.parallel_loop(0, KP, 1, unroll=2)
        def _row_fma(r):
            for j in range(VECS):
                sl = pl.ds(j * LANES, LANES)
                rv[r, sl] = rv[r, sl] * SCALE + pv[r, sl]

        pend_out[buf] = start_out(k)
        if b == BATCH - 1 and pc + 2 < NPC:
            pend_pe[pc % 2] = start_pe(pc + 2)

    pend_out[0].wait()
    pend_out[1].wait()


def kernel(tokens, table):
    idx = (tokens.astype(jnp.int32)
           .reshape(BATCH, NW, NPC, KP)
           .transpose(1, 2, 0, 3))
    pe = jnp.asarray(_PE)

    mesh = plsc.VectorSubcoreMesh(core_axis_name="c", subcore_axis_name="s")
    out = pl.kernel(
        _body,
        out_type=jax.ShapeDtypeStruct((BATCH * SEQ, D_MODEL), jnp.float32),
        mesh=mesh,
        scratch_types=[
            pltpu.VMEM((NPC, BATCH, KP), jnp.int32),
            pltpu.VMEM((KP, D_MODEL), jnp.float32),
            pltpu.VMEM((KP, D_MODEL), jnp.float32),
            pltpu.VMEM((KP, D_MODEL), jnp.float32),
            pltpu.VMEM((KP, D_MODEL), jnp.float32),
            pltpu.SemaphoreType.DMA,
            pltpu.SemaphoreType.DMA,
            pltpu.SemaphoreType.DMA,
            pltpu.SemaphoreType.DMA,
            pltpu.SemaphoreType.DMA,
            pltpu.SemaphoreType.DMA,
        ],
    )(idx, table, pe)
    return out.reshape(BATCH, SEQ, D_MODEL)
